# baseline (speedup 1.0000x reference)
import functools

import jax
import jax.numpy as jnp
from jax import lax
from jax.experimental import pallas as pl
from jax.experimental.pallas import tpu as pltpu
from jax.experimental.pallas import tpu_sc as plsc

N = 10000
D = 128
E = 320000
L = 100000

NPAD = 10240
CHUNK = 128
NCHUNKS = 2560
EPAD = NCHUNKS * CHUNK
CPW = NCHUNKS // 32
HCPW = CPW // 5
ROWS_PT = NPAD // 16

LPAD = 102400
PPW = LPAD // 32
PPP = 160
DCH = 32
DPH = PPW // PPP
CPP = PPP // DCH

_MESH = plsc.VectorSubcoreMesh(core_axis_name="c", subcore_axis_name="s")
_f32 = jnp.float32
_i32 = jnp.int32


def _scatter_body(x_hbm, src_hbm, dst_hbm, acc_out, degr_out,
                  src_v, dst_v, bufA, bufB, acc_sp, semA, semB):
    c = lax.axis_index("c")
    s = lax.axis_index("s")
    w = s * 2 + c
    bufs = (bufA, bufB)
    sems = (semA, semB)

    def fill_bufA(val):
        vec = jnp.full((16,), val, _f32)

        def frow(r, carry):
            for k in range(D // 16):
                bufA[r, pl.ds(k * 16, 16)] = vec
            return carry
        lax.fori_loop(0, CHUNK, frow, 0)

    for acc_out_ref, is_deg in ((acc_out, False), (degr_out, True)):
        fill_bufA(0.0)
        for t in range(ROWS_PT // CHUNK):
            r0 = s * ROWS_PT + t * CHUNK
            pltpu.sync_copy(bufA, acc_sp.at[pl.ds(r0, CHUNK)])
        if is_deg:
            fill_bufA(1.0)
        plsc.subcore_barrier()

        if is_deg:
            def deg_phase(ph, carry):
                pltpu.sync_copy(
                    dst_hbm.at[pl.ds(w * CPW + ph * HCPW, HCPW)], dst_v)

                def fire(j, carry2):
                    pltpu.async_copy(bufA, acc_sp.at[dst_v.at[j]],
                                     add=True, sem=semB)
                    return carry2
                lax.fori_loop(0, HCPW, fire, 0)

                def drain(j, carry2):
                    pltpu.make_async_copy(bufA, acc_sp.at[pl.ds(0, CHUNK)],
                                          semB).wait()
                    return carry2
                lax.fori_loop(0, HCPW, drain, 0)
                return carry
            lax.fori_loop(0, CPW // HCPW, deg_phase, 0)
        else:
            def gather_phase(ph, carry):
                base = w * CPW + ph * HCPW
                pltpu.sync_copy(src_hbm.at[pl.ds(base, HCPW)], src_v)
                pltpu.sync_copy(dst_hbm.at[pl.ds(base, HCPW)], dst_v)
                pltpu.async_copy(x_hbm.at[src_v.at[0]], bufA, semA)
                for k in range(HCPW):
                    if k + 1 < HCPW:
                        pltpu.async_copy(x_hbm.at[src_v.at[k + 1]],
                                         bufs[(k + 1) % 2], sems[(k + 1) % 2])
                    pltpu.make_async_copy(x_hbm.at[pl.ds(0, CHUNK)],
                                          bufs[k % 2], sems[k % 2]).wait()
                    pltpu.sync_copy(bufs[k % 2], acc_sp.at[dst_v.at[k]],
                                    add=True)
                return carry
            lax.fori_loop(0, CPW // HCPW, gather_phase, 0)
        plsc.subcore_barrier()

        for t in range(ROWS_PT // CHUNK):
            r0 = s * ROWS_PT + t * CHUNK
            pltpu.sync_copy(acc_sp.at[pl.ds(r0, CHUNK)], bufA)
            pltpu.sync_copy(bufA, acc_out_ref.at[c, pl.ds(r0, CHUNK)])


_sc_scatter = functools.partial(
    pl.kernel,
    out_type=(
        jax.ShapeDtypeStruct((2, NPAD, D), _f32),
        jax.ShapeDtypeStruct((2, NPAD, D), _f32),
    ),
    mesh=_MESH,
    scratch_types=[
        pltpu.VMEM((HCPW, CHUNK), _i32),
        pltpu.VMEM((HCPW, CHUNK), _i32),
        pltpu.VMEM((CHUNK, D), _f32),
        pltpu.VMEM((CHUNK, D), _f32),
        pltpu.VMEM_SHARED((NPAD, D), _f32),
        pltpu.SemaphoreType.DMA,
        pltpu.SemaphoreType.DMA,
    ],
    compiler_params=pltpu.CompilerParams(use_tc_tiling_on_sc=False),
)(_scatter_body)


def _dense_body(x_ref, aggp_ref, deg_ref,
                ws_ref, wn_ref, b_ref, g_ref, be_ref, flag_ref, out_ref):
    x = x_ref[...]
    deg = deg_ref[:N, :]
    inv = 1.0 / jnp.maximum(deg, 1.0)
    agg = aggp_ref[0, :N, :] + aggp_ref[1, :N, :]
    h = (jnp.dot(x, ws_ref[...], preferred_element_type=_f32)
         + jnp.dot(agg * inv, wn_ref[...], preferred_element_type=_f32)
         + b_ref[...])
    mu = jnp.mean(h, axis=0, keepdims=True)
    var = jnp.mean((h - mu) * (h - mu), axis=0, keepdims=True)
    hn = (h - mu) * lax.rsqrt(var + 1e-5) * g_ref[...] + be_ref[...]
    hn = jnp.where(flag_ref[...] > 0.0, jnp.where(hn >= 0.0, hn, 0.01 * hn),
                   hn)
    out_ref[...] = hn


def _dense_layer(x, aggp, deg_col, ws, wn, b, g, be, flag):
    return pl.pallas_call(
        _dense_body,
        out_shape=jax.ShapeDtypeStruct((N, D), _f32),
        compiler_params=pltpu.CompilerParams(
            vmem_limit_bytes=128 * 1024 * 1024),
    )(x, aggp, deg_col, ws, wn, b, g, be, flag)


def _decoder_body(h_hbm, i0_hbm, i1_hbm, pred_hbm,
                  i0_v, i1_v, bufA, bufB, outv, semA, semB):
    c = lax.axis_index("c")
    s = lax.axis_index("s")
    w = s * 2 + c

    def phase_body(ph, carry):
        base = w * PPW + ph * PPP
        pltpu.sync_copy(i0_hbm.at[pl.ds(base, PPP)], i0_v)
        pltpu.sync_copy(i1_hbm.at[pl.ds(base, PPP)], i1_v)

        def pair_chunk(j, carry2):
            cpA = pltpu.async_copy(h_hbm.at[i0_v.at[pl.ds(j * DCH, DCH)]],
                                   bufA, semA)
            cpB = pltpu.async_copy(h_hbm.at[i1_v.at[pl.ds(j * DCH, DCH)]],
                                   bufB, semB)
            cpA.wait()
            cpB.wait()

            def pair_dot(p, carry3):
                acc = bufA[p, pl.ds(0, 16)] * bufB[p, pl.ds(0, 16)]
                for k in range(1, D // 16):
                    acc = acc + (bufA[p, pl.ds(k * 16, 16)]
                                 * bufB[p, pl.ds(k * 16, 16)])
                outv[j * DCH + p, pl.ds(0, 16)] = acc
                return carry3
            lax.fori_loop(0, DCH, pair_dot, 0)
            return carry2
        lax.fori_loop(0, CPP, pair_chunk, 0)
        pltpu.sync_copy(outv, pred_hbm.at[pl.ds(base, PPP)])
        return carry
    lax.fori_loop(0, DPH, phase_body, 0)


_sc_decoder = functools.partial(
    pl.kernel,
    out_type=jax.ShapeDtypeStruct((LPAD, 16), _f32),
    mesh=_MESH,
    scratch_types=[
        pltpu.VMEM((PPP,), _i32),
        pltpu.VMEM((PPP,), _i32),
        pltpu.VMEM((DCH, D), _f32),
        pltpu.VMEM((DCH, D), _f32),
        pltpu.VMEM((PPP, 16), _f32),
        pltpu.SemaphoreType.DMA,
        pltpu.SemaphoreType.DMA,
    ],
)(_decoder_body)


def _final_reduce_body(p2_ref, sel_ref, out_ref):
    out_ref[...] = jnp.dot(p2_ref[...], sel_ref[...],
                           preferred_element_type=_f32)


def _final_reduce(p2, sel):
    return pl.pallas_call(
        _final_reduce_body,
        out_shape=jax.ShapeDtypeStruct((LPAD // 8, 8), _f32),
    )(p2.reshape(LPAD // 8, 128), sel)


def kernel(x, edge_index, edge_label_index,
           W1_self, W1_neigh, b1, gamma1, beta1,
           W2_self, W2_neigh, b2, gamma2, beta2):
    src_p = jnp.concatenate(
        [edge_index[0], jnp.zeros((EPAD - E,), _i32)]).reshape(NCHUNKS, CHUNK)
    dst_p = jnp.concatenate(
        [edge_index[1], jnp.full((EPAD - E,), N, _i32)]).reshape(NCHUNKS, CHUNK)
    i0_p = jnp.concatenate([edge_label_index[0], jnp.zeros((LPAD - L,), _i32)])
    i1_p = jnp.concatenate([edge_label_index[1], jnp.zeros((LPAD - L,), _i32)])

    w_self = jnp.stack([W1_self, W2_self])
    w_neigh = jnp.stack([W1_neigh, W2_neigh])
    bias = jnp.stack([b1.reshape(1, D), b2.reshape(1, D)])
    gamma = jnp.stack([gamma1.reshape(1, D), gamma2.reshape(1, D)])
    beta = jnp.stack([beta1.reshape(1, D), beta2.reshape(1, D)])
    lrelu_flag = jnp.array([[[1.0]], [[0.0]]], _f32)

    def layer(i, h):
        acc, degr = _sc_scatter(h, src_p, dst_p)
        deg_col = (degr[0, :, 0] + degr[1, :, 0]).reshape(NPAD, 1)
        ix = jnp.minimum(i, 1)
        return _dense_layer(h, acc, deg_col,
                            w_self[ix], w_neigh[ix],
                            bias[ix], gamma[ix], beta[ix], lrelu_flag[ix])

    n_layers = 2 + jnp.where(edge_index[0, 0] < 0, 1, 0).astype(_i32)
    h2 = lax.fori_loop(0, n_layers, layer, x)
    pred2 = _sc_decoder(h2, i0_p, i1_p)
    sel = jnp.repeat(jnp.eye(8, dtype=_f32), 16, axis=0)
    pred_p = _final_reduce(pred2, sel)
    return pred_p.reshape(LPAD)[:L]


# speedup vs baseline: 1.0100x; 1.0100x over previous
import functools

import jax
import jax.numpy as jnp
from jax import lax
from jax.experimental import pallas as pl
from jax.experimental.pallas import tpu as pltpu
from jax.experimental.pallas import tpu_sc as plsc

N = 10000
D = 128
QD = D // 4
E = 320000
L = 100000

NPAD = 10240
CHUNK = 128
NCHUNKS = 2560
EPAD = NCHUNKS * CHUNK
CPW = NCHUNKS // 32
ROWS_PT = NPAD // 16

LPAD = 102400
PPW = LPAD // 32
PPP = 640
DCH = 32
DPH = PPW // PPP
CPP = PPP // DCH

_MESH = plsc.VectorSubcoreMesh(core_axis_name="c", subcore_axis_name="s")
_f32 = jnp.float32
_i32 = jnp.int32


def _scatter_body(xa_hbm, xb_hbm, xc_hbm, xd_hbm, src_hbm, dst_hbm,
                  acca_out, accb_out, accc_out, accd_out, degr_out,
                  src_v, dst_v, bufA, bufB, acc_sp, semA, semB):
    c = lax.axis_index("c")
    s = lax.axis_index("s")
    w = s * 2 + c

    def fill_buf(buf, val):
        vec = jnp.full((16,), val, _f32)

        def frow(r, carry):
            for k in range(QD // 16):
                buf[r, pl.ds(k * 16, 16)] = vec
            return carry
        lax.fori_loop(0, CHUNK, frow, 0)

    pltpu.sync_copy(src_hbm.at[pl.ds(w * CPW, CPW)], src_v)
    pltpu.sync_copy(dst_hbm.at[pl.ds(w * CPW, CPW)], dst_v)

    for x_hbm, acc_out in ((xa_hbm, acca_out), (xb_hbm, accb_out),
                           (xc_hbm, accc_out), (xd_hbm, accd_out),
                           (None, degr_out)):
        is_deg = x_hbm is None
        fill_buf(bufA, 0.0)
        for t in range(ROWS_PT // CHUNK):
            r0 = s * ROWS_PT + t * CHUNK
            pltpu.sync_copy(bufA, acc_sp.at[pl.ds(r0, CHUNK)])
        if is_deg:
            fill_buf(bufA, 1.0)
        plsc.subcore_barrier()

        if is_deg:
            def deg_chunk(j, carry):
                pltpu.async_copy(bufA, acc_sp.at[dst_v.at[j]],
                                 add=True, sem=semB)
                return carry
            lax.fori_loop(0, CPW, deg_chunk, 0)

            def deg_drain(j, carry):
                pltpu.make_async_copy(bufA, acc_sp.at[pl.ds(0, CHUNK)],
                                      semB).wait()
                return carry
            lax.fori_loop(0, CPW, deg_drain, 0)
        else:
            pltpu.async_copy(x_hbm.at[src_v.at[0]], bufA, semA)

            def edge_pair(j, carry):
                pltpu.async_copy(x_hbm.at[src_v.at[2 * j + 1]], bufB, semB)
                pltpu.make_async_copy(x_hbm.at[pl.ds(0, CHUNK)], bufA,
                                      semA).wait()
                pltpu.sync_copy(bufA, acc_sp.at[dst_v.at[2 * j]], add=True)

                @pl.when(2 * j + 2 < CPW)
                def _():
                    pltpu.async_copy(x_hbm.at[src_v.at[2 * j + 2]], bufA,
                                     semA)
                pltpu.make_async_copy(x_hbm.at[pl.ds(0, CHUNK)], bufB,
                                      semB).wait()
                pltpu.sync_copy(bufB, acc_sp.at[dst_v.at[2 * j + 1]],
                                add=True)
                return carry
            lax.fori_loop(0, CPW // 2, edge_pair, 0)
        plsc.subcore_barrier()

        for t in range(ROWS_PT // CHUNK):
            r0 = s * ROWS_PT + t * CHUNK
            pltpu.sync_copy(acc_sp.at[pl.ds(r0, CHUNK)], bufA)
            pltpu.sync_copy(bufA, acc_out.at[c, pl.ds(r0, CHUNK)])


_sc_scatter = functools.partial(
    pl.kernel,
    out_type=tuple(jax.ShapeDtypeStruct((2, NPAD, QD), _f32)
                   for _ in range(5)),
    mesh=_MESH,
    scratch_types=[
        pltpu.VMEM((CPW, CHUNK), _i32),
        pltpu.VMEM((CPW, CHUNK), _i32),
        pltpu.VMEM((CHUNK, QD), _f32),
        pltpu.VMEM((CHUNK, QD), _f32),
        pltpu.VMEM_SHARED((NPAD, QD), _f32),
        pltpu.SemaphoreType.DMA,
        pltpu.SemaphoreType.DMA,
    ],
    compiler_params=pltpu.CompilerParams(use_tc_tiling_on_sc=False),
)(_scatter_body)


def _dense_body(x_ref, aggp_ref, deg_ref,
                ws_ref, wn_ref, b_ref, g_ref, be_ref, flag_ref, out_ref):
    x = x_ref[...]
    deg = deg_ref[:N, :]
    inv = 1.0 / jnp.maximum(deg, 1.0)
    agg = aggp_ref[0, :N, :] + aggp_ref[1, :N, :]
    h = (jnp.dot(x, ws_ref[...], preferred_element_type=_f32)
         + jnp.dot(agg * inv, wn_ref[...], preferred_element_type=_f32)
         + b_ref[...])
    mu = jnp.mean(h, axis=0, keepdims=True)
    var = jnp.mean((h - mu) * (h - mu), axis=0, keepdims=True)
    hn = (h - mu) * lax.rsqrt(var + 1e-5) * g_ref[...] + be_ref[...]
    hn = jnp.where(flag_ref[...] > 0.0, jnp.where(hn >= 0.0, hn, 0.01 * hn),
                   hn)
    out_ref[...] = hn


def _dense_layer(x, aggp, deg_col, ws, wn, b, g, be, flag):
    return pl.pallas_call(
        _dense_body,
        out_shape=jax.ShapeDtypeStruct((N, D), _f32),
        compiler_params=pltpu.CompilerParams(
            vmem_limit_bytes=128 * 1024 * 1024),
    )(x, aggp, deg_col, ws, wn, b, g, be, flag)


def _decoder_body(h_hbm, i0_hbm, i1_hbm, pred_hbm,
                  i0_v, i1_v, bufA0, bufB0, bufA1, bufB1, outv,
                  semA0, semB0, semA1, semB1):
    c = lax.axis_index("c")
    s = lax.axis_index("s")
    w = s * 2 + c

    def start(j, bufA, bufB, semA, semB):
        pltpu.async_copy(h_hbm.at[i0_v.at[pl.ds(j * DCH, DCH)]], bufA, semA)
        pltpu.async_copy(h_hbm.at[i1_v.at[pl.ds(j * DCH, DCH)]], bufB, semB)

    def finish(j, bufA, bufB, semA, semB):
        dummy = h_hbm.at[pl.ds(0, DCH)]
        pltpu.make_async_copy(dummy, bufA, semA).wait()
        pltpu.make_async_copy(dummy, bufB, semB).wait()

        def pair_dot(p, carry3):
            acc = bufA[p, pl.ds(0, 16)] * bufB[p, pl.ds(0, 16)]
            for k in range(1, D // 16):
                acc = acc + (bufA[p, pl.ds(k * 16, 16)]
                             * bufB[p, pl.ds(k * 16, 16)])
            outv[j * DCH + p, pl.ds(0, 16)] = acc
            return carry3
        lax.fori_loop(0, DCH, pair_dot, 0)

    def phase_body(ph, carry):
        base = w * PPW + ph * PPP
        pltpu.sync_copy(i0_hbm.at[pl.ds(base, PPP)], i0_v)
        pltpu.sync_copy(i1_hbm.at[pl.ds(base, PPP)], i1_v)
        start(0, bufA0, bufB0, semA0, semB0)

        def chunk_pair(j, carry2):
            start(2 * j + 1, bufA1, bufB1, semA1, semB1)
            finish(2 * j, bufA0, bufB0, semA0, semB0)

            @pl.when(2 * j + 2 < CPP)
            def _():
                start(2 * j + 2, bufA0, bufB0, semA0, semB0)
            finish(2 * j + 1, bufA1, bufB1, semA1, semB1)
            return carry2
        lax.fori_loop(0, CPP // 2, chunk_pair, 0)
        pltpu.sync_copy(outv, pred_hbm.at[pl.ds(base, PPP)])
        return carry
    lax.fori_loop(0, DPH, phase_body, 0)


_sc_decoder = functools.partial(
    pl.kernel,
    out_type=jax.ShapeDtypeStruct((LPAD, 16), _f32),
    mesh=_MESH,
    scratch_types=[
        pltpu.VMEM((PPP,), _i32),
        pltpu.VMEM((PPP,), _i32),
        pltpu.VMEM((DCH, D), _f32),
        pltpu.VMEM((DCH, D), _f32),
        pltpu.VMEM((DCH, D), _f32),
        pltpu.VMEM((DCH, D), _f32),
        pltpu.VMEM((PPP, 16), _f32),
        pltpu.SemaphoreType.DMA,
        pltpu.SemaphoreType.DMA,
        pltpu.SemaphoreType.DMA,
        pltpu.SemaphoreType.DMA,
    ],
)(_decoder_body)


def _final_reduce_body(p2_ref, sel_ref, out_ref):
    out_ref[...] = jnp.dot(p2_ref[...], sel_ref[...],
                           preferred_element_type=_f32)


def _final_reduce(p2, sel):
    return pl.pallas_call(
        _final_reduce_body,
        out_shape=jax.ShapeDtypeStruct((LPAD // 8, 8), _f32),
    )(p2.reshape(LPAD // 8, 128), sel)


def kernel(x, edge_index, edge_label_index,
           W1_self, W1_neigh, b1, gamma1, beta1,
           W2_self, W2_neigh, b2, gamma2, beta2):
    src_p = jnp.concatenate(
        [edge_index[0], jnp.zeros((EPAD - E,), _i32)]).reshape(NCHUNKS, CHUNK)
    dst_p = jnp.concatenate(
        [edge_index[1], jnp.full((EPAD - E,), N, _i32)]).reshape(NCHUNKS, CHUNK)
    i0_p = jnp.concatenate([edge_label_index[0], jnp.zeros((LPAD - L,), _i32)])
    i1_p = jnp.concatenate([edge_label_index[1], jnp.zeros((LPAD - L,), _i32)])

    w_self = jnp.stack([W1_self, W2_self])
    w_neigh = jnp.stack([W1_neigh, W2_neigh])
    bias = jnp.stack([b1.reshape(1, D), b2.reshape(1, D)])
    gamma = jnp.stack([gamma1.reshape(1, D), gamma2.reshape(1, D)])
    beta = jnp.stack([beta1.reshape(1, D), beta2.reshape(1, D)])
    lrelu_flag = jnp.array([[[1.0]], [[0.0]]], _f32)

    def layer(i, h):
        xq = [lax.slice(h, (0, q * QD), (N, (q + 1) * QD)) for q in range(4)]
        acca, accb, accc, accd, degr = _sc_scatter(
            xq[0], xq[1], xq[2], xq[3], src_p, dst_p)
        aggp = jnp.concatenate([acca, accb, accc, accd], axis=2)
        deg_col = (degr[0, :, 0] + degr[1, :, 0]).reshape(NPAD, 1)
        return _dense_layer(h, aggp, deg_col,
                            w_self[i], w_neigh[i],
                            bias[i], gamma[i], beta[i], lrelu_flag[i])

    h2 = lax.fori_loop(0, 2, layer, x)
    pred2 = _sc_decoder(h2, i0_p, i1_p)
    sel = jnp.repeat(jnp.eye(8, dtype=_f32), 16, axis=0)
    pred_p = _final_reduce(pred2, sel)
    return pred_p.reshape(LPAD)[:L]
